# baseline (speedup 1.0000x reference)
import functools

import jax
import jax.numpy as jnp
from jax import lax
from jax.experimental import pallas as pl
from jax.experimental.pallas import tpu as pltpu
from jax.experimental.pallas import tpu_sc as plsc

N = 10000
E = 320000
DIN = 128
DH = 256

NSUB = 16
NCORE = 2
NW = NCORE * NSUB
CHUNK = 128
EP = 327680
NPAD = EP - E
NDUM = 128
NCH1 = EP // NW // CHUNK
NCH2 = EP // NSUB // CHUNK
IB1 = 16
NB1 = NCH1 // IB1
IB2 = 32
NB2 = NCH2 // IB2
ZROWS = 624
ZTAIL = N - NSUB * ZROWS

_mesh = plsc.VectorSubcoreMesh(core_axis_name="c", subcore_axis_name="s")


def _copy_row_slices(src_at, dst_at, s):
    pltpu.sync_copy(src_at(pl.ds(s * ZROWS, ZROWS)),
                    dst_at(pl.ds(s * ZROWS, ZROWS)))

    @pl.when(s == 0)
    def _():
        pltpu.sync_copy(src_at(pl.ds(NSUB * ZROWS, ZTAIL)),
                        dst_at(pl.ds(NSUB * ZROWS, ZTAIL)))


@functools.partial(
    pl.kernel,
    mesh=_mesh,
    out_type=(
        jax.ShapeDtypeStruct((NCORE, N, DIN), jnp.float32),
        jax.ShapeDtypeStruct((NCORE, N + NDUM), jnp.float32),
    ),
    scratch_types=[
        pltpu.VMEM((IB1, CHUNK), jnp.int32),
        pltpu.VMEM((IB1, CHUNK), jnp.int32),
        pltpu.VMEM((CHUNK, DIN), jnp.float32),
        pltpu.VMEM((CHUNK, DIN), jnp.float32),
        pltpu.VMEM((CHUNK,), jnp.float32),
        pltpu.VMEM_SHARED((N + NDUM, DIN), jnp.float32),
        pltpu.VMEM_SHARED((N + NDUM,), jnp.float32),
        pltpu.SemaphoreType.DMA,
        pltpu.SemaphoreType.DMA,
        pltpu.SemaphoreType.DMA,
        pltpu.SemaphoreType.DMA,
        pltpu.SemaphoreType.DMA,
        pltpu.SemaphoreType.DMA,
    ],
)
def _sc_agg1(x_hbm, src_hbm, dst_hbm, zf_hbm, zc_hbm,
             agg_hbm, cnt_hbm,
             src_v, dst_v, rows0, rows1, ones_v, acc_sh, cnt_sh,
             gs0, gs1, ss0, ss1, cs0, cs1):
    c = lax.axis_index("c")
    s = lax.axis_index("s")
    wid = c * NSUB + s
    rows = (rows0, rows1)
    gs = (gs0, gs1)
    ss = (ss0, ss1)
    cs = (cs0, cs1)
    for j in range(CHUNK // 16):
        ones_v[pl.ds(j * 16, 16)] = jnp.ones((16,), jnp.float32)
    _copy_row_slices(lambda d: zf_hbm.at[d], lambda d: acc_sh.at[d], s)

    @pl.when(s == 0)
    def _():
        pltpu.sync_copy(zc_hbm, cnt_sh)

    plsc.subcore_barrier()

    def g_start(k, b):
        pltpu.async_copy(x_hbm.at[src_v.at[k]], rows[b], gs[b])

    def g_wait(k, b):
        pltpu.make_async_copy(x_hbm.at[src_v.at[k]], rows[b], gs[b]).wait()

    def s_start(k, b):
        pltpu.async_copy(rows[b], acc_sh.at[dst_v.at[k]], ss[b], add=True)
        pltpu.async_copy(ones_v, cnt_sh.at[dst_v.at[k]], cs[b], add=True)

    def s_wait(k, b):
        pltpu.make_async_copy(rows[b], acc_sh.at[dst_v.at[k]], ss[b]).wait()
        pltpu.make_async_copy(ones_v, cnt_sh.at[dst_v.at[k]], cs[b]).wait()

    def block_body(blk, carry):
        pltpu.sync_copy(src_hbm.at[wid * NB1 + blk], src_v)
        pltpu.sync_copy(dst_hbm.at[wid * NB1 + blk], dst_v)
        g_start(0, 0)
        g_start(1, 1)

        def chunk_body(k2, carry2):
            k = 2 * k2
            g_wait(k, 0)
            s_start(k, 0)
            g_wait(k + 1, 1)
            s_start(k + 1, 1)
            s_wait(k, 0)
            g_start(k + 2, 0)
            s_wait(k + 1, 1)
            g_start(k + 3, 1)
            return carry2

        lax.fori_loop(0, IB1 // 2 - 1, chunk_body, 0)
        k = IB1 - 2
        g_wait(k, 0)
        s_start(k, 0)
        g_wait(k + 1, 1)
        s_start(k + 1, 1)
        s_wait(k, 0)
        s_wait(k + 1, 1)
        return carry

    lax.fori_loop(0, NB1, block_body, 0)

    plsc.subcore_barrier()
    _copy_row_slices(lambda d: acc_sh.at[d], lambda d: agg_hbm.at[c, d], s)

    @pl.when(s == 0)
    def _():
        pltpu.sync_copy(cnt_sh, cnt_hbm.at[c])


@functools.partial(
    pl.kernel,
    mesh=_mesh,
    out_type=jax.ShapeDtypeStruct((NCORE, N, DIN), jnp.float32),
    scratch_types=[
        pltpu.VMEM((IB2, CHUNK), jnp.int32),
        pltpu.VMEM((IB2, CHUNK), jnp.int32),
        pltpu.VMEM((CHUNK, DIN), jnp.float32),
        pltpu.VMEM((CHUNK, DIN), jnp.float32),
        pltpu.VMEM_SHARED((N + NDUM, DIN), jnp.float32),
        pltpu.SemaphoreType.DMA,
        pltpu.SemaphoreType.DMA,
        pltpu.SemaphoreType.DMA,
        pltpu.SemaphoreType.DMA,
    ],
)
def _sc_agg2(h1s_hbm, srcoff_hbm, dst_hbm, zf_hbm,
             agg_hbm,
             src_v, dst_v, rows0, rows1, acc_sh,
             gs0, gs1, ss0, ss1):
    c = lax.axis_index("c")
    s = lax.axis_index("s")
    wid = c * NSUB + s
    rows = (rows0, rows1)
    gs = (gs0, gs1)
    ss = (ss0, ss1)
    _copy_row_slices(lambda d: zf_hbm.at[d], lambda d: acc_sh.at[d], s)
    plsc.subcore_barrier()

    def g_start(k, b):
        pltpu.async_copy(h1s_hbm.at[src_v.at[k]], rows[b], gs[b])

    def g_wait(k, b):
        pltpu.make_async_copy(h1s_hbm.at[src_v.at[k]], rows[b], gs[b]).wait()

    def s_start(k, b):
        pltpu.async_copy(rows[b], acc_sh.at[dst_v.at[k]], ss[b], add=True)

    def s_wait(k, b):
        pltpu.make_async_copy(rows[b], acc_sh.at[dst_v.at[k]], ss[b]).wait()

    def block_body(blk, carry):
        pltpu.sync_copy(srcoff_hbm.at[wid * NB2 + blk], src_v)
        pltpu.sync_copy(dst_hbm.at[s * NB2 + blk], dst_v)
        g_start(0, 0)
        g_start(1, 1)

        def chunk_body(k2, carry2):
            k = 2 * k2
            g_wait(k, 0)
            s_start(k, 0)
            g_wait(k + 1, 1)
            s_start(k + 1, 1)
            s_wait(k, 0)
            g_start(k + 2, 0)
            s_wait(k + 1, 1)
            g_start(k + 3, 1)
            return carry2

        lax.fori_loop(0, IB2 // 2 - 1, chunk_body, 0)
        k = IB2 - 2
        g_wait(k, 0)
        s_start(k, 0)
        g_wait(k + 1, 1)
        s_start(k + 1, 1)
        s_wait(k, 0)
        s_wait(k + 1, 1)
        return carry

    lax.fori_loop(0, NB2, block_body, 0)

    plsc.subcore_barrier()
    _copy_row_slices(lambda d: acc_sh.at[d], lambda d: agg_hbm.at[c, d], s)


def _tc1_body(x_ref, aggp_ref, cnt_ref, wl_ref, wr_ref, b_ref,
              h1_ref, h1s_ref):
    agg = aggp_ref[0] + aggp_ref[1]
    cnt = cnt_ref[:, 0] + cnt_ref[:, 1]
    inv = 1.0 / jnp.maximum(cnt, 1.0)
    aggn = agg * inv[:, None]
    h = (jnp.dot(aggn, wl_ref[...], preferred_element_type=jnp.float32)
         + jnp.dot(x_ref[...], wr_ref[...], preferred_element_type=jnp.float32)
         + b_ref[...])
    h = jnp.maximum(h, 0.0)
    h1_ref[...] = h
    h1s_ref[0] = h[:, :DIN]
    h1s_ref[1] = h[:, DIN:]


def _tc2_body(h1_ref, aggp_ref, cnt_ref, wl_ref, wr_ref, b_ref,
              w3_ref, b3_ref, h2_ref, out_ref):
    cnt = cnt_ref[:, 0] + cnt_ref[:, 1]
    inv = 1.0 / jnp.maximum(cnt, 1.0)
    a0 = aggp_ref[0] * inv[:, None]
    a1 = aggp_ref[1] * inv[:, None]
    wl = wl_ref[...]
    h2 = (jnp.dot(a0, wl[:DIN], preferred_element_type=jnp.float32)
          + jnp.dot(a1, wl[DIN:], preferred_element_type=jnp.float32)
          + jnp.dot(h1_ref[...], wr_ref[...],
                    preferred_element_type=jnp.float32)
          + b_ref[...])
    h2_ref[...] = h2
    out_ref[...] = (jnp.dot(h2, w3_ref[...], preferred_element_type=jnp.float32)
                    + b3_ref[...])


def kernel(x, edge_index, W1l, W1r, b1, W2l, W2r, b2, W3, b3):
    src = edge_index[0].astype(jnp.int32)
    dst = edge_index[1].astype(jnp.int32)
    zpad = (jnp.arange(NPAD, dtype=jnp.int32) * 131) % N
    dpad = N + (jnp.arange(NPAD, dtype=jnp.int32) % NDUM)
    srcp = jnp.concatenate([src, zpad])
    dstp = jnp.concatenate([dst, dpad])
    src1 = srcp.reshape(NW * NB1, IB1, CHUNK)
    dst1 = dstp.reshape(NW * NB1, IB1, CHUNK)
    srcoff = jnp.concatenate([src, zpad, src + N, zpad]).reshape(
        NW * NB2, IB2, CHUNK)
    dst2 = dstp.reshape(NSUB * NB2, IB2, CHUNK)
    zf = jnp.zeros((N, DIN), jnp.float32)
    zc = jnp.zeros((N + NDUM,), jnp.float32)

    aggp1, cntp = _sc_agg1(x, src1, dst1, zf, zc)
    cnt_t = cntp[:, :N].T

    R = 400
    grid = (N // R,)
    h1, h1s = pl.pallas_call(
        _tc1_body,
        grid=grid,
        in_specs=[
            pl.BlockSpec((R, DIN), lambda i: (i, 0)),
            pl.BlockSpec((NCORE, R, DIN), lambda i: (0, i, 0)),
            pl.BlockSpec((R, NCORE), lambda i: (i, 0)),
            pl.BlockSpec((DIN, DH), lambda i: (0, 0)),
            pl.BlockSpec((DIN, DH), lambda i: (0, 0)),
            pl.BlockSpec((1, DH), lambda i: (0, 0)),
        ],
        out_specs=[
            pl.BlockSpec((R, DH), lambda i: (i, 0)),
            pl.BlockSpec((NCORE, R, DIN), lambda i: (0, i, 0)),
        ],
        out_shape=[
            jax.ShapeDtypeStruct((N, DH), jnp.float32),
            jax.ShapeDtypeStruct((NCORE, N, DIN), jnp.float32),
        ],
    )(x, aggp1, cnt_t, W1l.T, W1r.T, b1[None, :])

    aggp2 = _sc_agg2(h1s.reshape(NCORE * N, DIN), srcoff, dst2, zf)

    h2, outc = pl.pallas_call(
        _tc2_body,
        grid=grid,
        in_specs=[
            pl.BlockSpec((R, DH), lambda i: (i, 0)),
            pl.BlockSpec((NCORE, R, DIN), lambda i: (0, i, 0)),
            pl.BlockSpec((R, NCORE), lambda i: (i, 0)),
            pl.BlockSpec((DH, DH), lambda i: (0, 0)),
            pl.BlockSpec((DH, DH), lambda i: (0, 0)),
            pl.BlockSpec((1, DH), lambda i: (0, 0)),
            pl.BlockSpec((DH, 1), lambda i: (0, 0)),
            pl.BlockSpec((1, 1), lambda i: (0, 0)),
        ],
        out_specs=[
            pl.BlockSpec((R, DH), lambda i: (i, 0)),
            pl.BlockSpec((R, 1), lambda i: (i, 0)),
        ],
        out_shape=[
            jax.ShapeDtypeStruct((N, DH), jnp.float32),
            jax.ShapeDtypeStruct((N, 1), jnp.float32),
        ],
    )(h1, aggp2, cnt_t, W2l.T, W2r.T, b2[None, :], W3.T, b3[None, :])

    return (outc[:, 0], h1, h2)


# speedup vs baseline: 1.2097x; 1.2097x over previous
import functools

import jax
import jax.numpy as jnp
from jax import lax
from jax.experimental import pallas as pl
from jax.experimental.pallas import tpu as pltpu
from jax.experimental.pallas import tpu_sc as plsc

N = 10000
E = 320000
DIN = 128
DH = 256

NSUB = 16
NCORE = 2
NW = NCORE * NSUB
CHUNK = 128
EP = 327680
NPAD = EP - E
NDUM = 128
NCH1 = EP // NW // CHUNK
NCH2 = EP // NSUB // CHUNK
IB1 = 16
NB1 = NCH1 // IB1
IB2 = 32
NB2 = NCH2 // IB2
ZROWS = 624
ZTAIL = N - NSUB * ZROWS

_mesh = plsc.VectorSubcoreMesh(core_axis_name="c", subcore_axis_name="s")


def _copy_row_slices(src_at, dst_at, s):
    pltpu.sync_copy(src_at(pl.ds(s * ZROWS, ZROWS)),
                    dst_at(pl.ds(s * ZROWS, ZROWS)))

    @pl.when(s == 0)
    def _():
        pltpu.sync_copy(src_at(pl.ds(NSUB * ZROWS, ZTAIL)),
                        dst_at(pl.ds(NSUB * ZROWS, ZTAIL)))


@functools.partial(
    pl.kernel,
    mesh=_mesh,
    out_type=(
        jax.ShapeDtypeStruct((NCORE, N, DIN), jnp.float32),
        jax.ShapeDtypeStruct((NCORE, N + NDUM), jnp.float32),
    ),
    scratch_types=[
        pltpu.VMEM((IB1, CHUNK), jnp.int32),
        pltpu.VMEM((IB1, CHUNK), jnp.int32),
        pltpu.VMEM((CHUNK, DIN), jnp.float32),
        pltpu.VMEM((CHUNK, DIN), jnp.float32),
        pltpu.VMEM((CHUNK,), jnp.float32),
        pltpu.VMEM_SHARED((N + NDUM, DIN), jnp.float32),
        pltpu.VMEM_SHARED((N + NDUM,), jnp.float32),
        pltpu.SemaphoreType.DMA,
        pltpu.SemaphoreType.DMA,
        pltpu.SemaphoreType.DMA,
    ],
)
def _sc_agg1(x_hbm, src_hbm, dst_hbm, zf_hbm, zc_hbm,
             agg_hbm, cnt_hbm,
             src_v, dst_v, rows0, rows1, ones_v, acc_sh, cnt_sh,
             gs0, gs1, csem):
    c = lax.axis_index("c")
    s = lax.axis_index("s")
    wid = c * NSUB + s
    rows = (rows0, rows1)
    gs = (gs0, gs1)
    for j in range(CHUNK // 16):
        ones_v[pl.ds(j * 16, 16)] = jnp.ones((16,), jnp.float32)
    _copy_row_slices(lambda d: zf_hbm.at[d], lambda d: acc_sh.at[d], s)

    @pl.when(s == 0)
    def _():
        pltpu.sync_copy(zc_hbm, cnt_sh)

    plsc.subcore_barrier()

    def g_start(k, b):
        pltpu.async_copy(x_hbm.at[src_v.at[k]], rows[b], gs[b])

    def g_wait(k, b):
        pltpu.make_async_copy(x_hbm.at[src_v.at[k]], rows[b], gs[b]).wait()

    def s_sync(k, b):
        pltpu.sync_copy(rows[b], acc_sh.at[dst_v.at[k]], add=True)
        pltpu.async_copy(ones_v, cnt_sh.at[dst_v.at[k]], csem, add=True)

    def c_drain(k):
        pltpu.make_async_copy(ones_v, cnt_sh.at[dst_v.at[k]], csem).wait()

    def block_body(blk, carry):
        pltpu.sync_copy(src_hbm.at[wid * NB1 + blk], src_v)
        pltpu.sync_copy(dst_hbm.at[wid * NB1 + blk], dst_v)
        g_start(0, 0)
        g_start(1, 1)

        def chunk_body(k2, carry2):
            k = 2 * k2
            g_wait(k, 0)
            s_sync(k, 0)
            g_start(k + 2, 0)
            g_wait(k + 1, 1)
            s_sync(k + 1, 1)
            g_start(k + 3, 1)
            return carry2

        lax.fori_loop(0, IB1 // 2 - 1, chunk_body, 0)
        k = IB1 - 2
        g_wait(k, 0)
        s_sync(k, 0)
        g_wait(k + 1, 1)
        s_sync(k + 1, 1)
        lax.fori_loop(0, IB1, lambda j, c2: (c_drain(j), c2)[1], 0)
        return carry

    lax.fori_loop(0, NB1, block_body, 0)

    plsc.subcore_barrier()
    _copy_row_slices(lambda d: acc_sh.at[d], lambda d: agg_hbm.at[c, d], s)

    @pl.when(s == 0)
    def _():
        pltpu.sync_copy(cnt_sh, cnt_hbm.at[c])


@functools.partial(
    pl.kernel,
    mesh=_mesh,
    out_type=jax.ShapeDtypeStruct((NCORE, N, DIN), jnp.float32),
    scratch_types=[
        pltpu.VMEM((IB2, CHUNK), jnp.int32),
        pltpu.VMEM((IB2, CHUNK), jnp.int32),
        pltpu.VMEM((CHUNK, DIN), jnp.float32),
        pltpu.VMEM((CHUNK, DIN), jnp.float32),
        pltpu.VMEM_SHARED((N + NDUM, DIN), jnp.float32),
        pltpu.SemaphoreType.DMA,
        pltpu.SemaphoreType.DMA,
    ],
)
def _sc_agg2(h1s_hbm, srcoff_hbm, dst_hbm, zf_hbm,
             agg_hbm,
             src_v, dst_v, rows0, rows1, acc_sh,
             gs0, gs1):
    c = lax.axis_index("c")
    s = lax.axis_index("s")
    wid = c * NSUB + s
    rows = (rows0, rows1)
    gs = (gs0, gs1)
    _copy_row_slices(lambda d: zf_hbm.at[d], lambda d: acc_sh.at[d], s)
    plsc.subcore_barrier()

    def g_start(k, b):
        pltpu.async_copy(h1s_hbm.at[src_v.at[k]], rows[b], gs[b])

    def g_wait(k, b):
        pltpu.make_async_copy(h1s_hbm.at[src_v.at[k]], rows[b], gs[b]).wait()

    def s_sync(k, b):
        pltpu.sync_copy(rows[b], acc_sh.at[dst_v.at[k]], add=True)

    def block_body(blk, carry):
        pltpu.sync_copy(srcoff_hbm.at[wid * NB2 + blk], src_v)
        pltpu.sync_copy(dst_hbm.at[s * NB2 + blk], dst_v)
        g_start(0, 0)
        g_start(1, 1)

        def chunk_body(k2, carry2):
            k = 2 * k2
            g_wait(k, 0)
            s_sync(k, 0)
            g_start(k + 2, 0)
            g_wait(k + 1, 1)
            s_sync(k + 1, 1)
            g_start(k + 3, 1)
            return carry2

        lax.fori_loop(0, IB2 // 2 - 1, chunk_body, 0)
        k = IB2 - 2
        g_wait(k, 0)
        s_sync(k, 0)
        g_wait(k + 1, 1)
        s_sync(k + 1, 1)
        return carry

    lax.fori_loop(0, NB2, block_body, 0)

    plsc.subcore_barrier()
    _copy_row_slices(lambda d: acc_sh.at[d], lambda d: agg_hbm.at[c, d], s)


def _tc1_body(x_ref, aggp_ref, cnt_ref, wl_ref, wr_ref, b_ref,
              h1_ref, h1s_ref):
    agg = aggp_ref[0] + aggp_ref[1]
    cnt = cnt_ref[:, 0] + cnt_ref[:, 1]
    inv = 1.0 / jnp.maximum(cnt, 1.0)
    aggn = agg * inv[:, None]
    h = (jnp.dot(aggn, wl_ref[...], preferred_element_type=jnp.float32)
         + jnp.dot(x_ref[...], wr_ref[...], preferred_element_type=jnp.float32)
         + b_ref[...])
    h = jnp.maximum(h, 0.0)
    h1_ref[...] = h
    h1s_ref[0] = h[:, :DIN]
    h1s_ref[1] = h[:, DIN:]


def _tc2_body(h1_ref, aggp_ref, cnt_ref, wl_ref, wr_ref, b_ref,
              w3_ref, b3_ref, h2_ref, out_ref):
    cnt = cnt_ref[:, 0] + cnt_ref[:, 1]
    inv = 1.0 / jnp.maximum(cnt, 1.0)
    a0 = aggp_ref[0] * inv[:, None]
    a1 = aggp_ref[1] * inv[:, None]
    wl = wl_ref[...]
    h2 = (jnp.dot(a0, wl[:DIN], preferred_element_type=jnp.float32)
          + jnp.dot(a1, wl[DIN:], preferred_element_type=jnp.float32)
          + jnp.dot(h1_ref[...], wr_ref[...],
                    preferred_element_type=jnp.float32)
          + b_ref[...])
    h2_ref[...] = h2
    out_ref[...] = (jnp.dot(h2, w3_ref[...], preferred_element_type=jnp.float32)
                    + b3_ref[...])


def kernel(x, edge_index, W1l, W1r, b1, W2l, W2r, b2, W3, b3):
    src = edge_index[0].astype(jnp.int32)
    dst = edge_index[1].astype(jnp.int32)
    zpad = (jnp.arange(NPAD, dtype=jnp.int32) * 131) % N
    dpad = N + (jnp.arange(NPAD, dtype=jnp.int32) % NDUM)
    srcp = jnp.concatenate([src, zpad])
    dstp = jnp.concatenate([dst, dpad])
    src1 = srcp.reshape(NW * NB1, IB1, CHUNK)
    dst1 = dstp.reshape(NW * NB1, IB1, CHUNK)
    srcoff = jnp.concatenate([src, zpad, src + N, zpad]).reshape(
        NW * NB2, IB2, CHUNK)
    dst2 = dstp.reshape(NSUB * NB2, IB2, CHUNK)
    zf = jnp.zeros((N, DIN), jnp.float32)
    zc = jnp.zeros((N + NDUM,), jnp.float32)

    aggp1, cntp = _sc_agg1(x, src1, dst1, zf, zc)
    cnt_t = cntp[:, :N].T

    R = 400
    grid = (N // R,)
    h1, h1s = pl.pallas_call(
        _tc1_body,
        grid=grid,
        in_specs=[
            pl.BlockSpec((R, DIN), lambda i: (i, 0)),
            pl.BlockSpec((NCORE, R, DIN), lambda i: (0, i, 0)),
            pl.BlockSpec((R, NCORE), lambda i: (i, 0)),
            pl.BlockSpec((DIN, DH), lambda i: (0, 0)),
            pl.BlockSpec((DIN, DH), lambda i: (0, 0)),
            pl.BlockSpec((1, DH), lambda i: (0, 0)),
        ],
        out_specs=[
            pl.BlockSpec((R, DH), lambda i: (i, 0)),
            pl.BlockSpec((NCORE, R, DIN), lambda i: (0, i, 0)),
        ],
        out_shape=[
            jax.ShapeDtypeStruct((N, DH), jnp.float32),
            jax.ShapeDtypeStruct((NCORE, N, DIN), jnp.float32),
        ],
    )(x, aggp1, cnt_t, W1l.T, W1r.T, b1[None, :])

    aggp2 = _sc_agg2(h1s.reshape(NCORE * N, DIN), srcoff, dst2, zf)

    h2, outc = pl.pallas_call(
        _tc2_body,
        grid=grid,
        in_specs=[
            pl.BlockSpec((R, DH), lambda i: (i, 0)),
            pl.BlockSpec((NCORE, R, DIN), lambda i: (0, i, 0)),
            pl.BlockSpec((R, NCORE), lambda i: (i, 0)),
            pl.BlockSpec((DH, DH), lambda i: (0, 0)),
            pl.BlockSpec((DH, DH), lambda i: (0, 0)),
            pl.BlockSpec((1, DH), lambda i: (0, 0)),
            pl.BlockSpec((DH, 1), lambda i: (0, 0)),
            pl.BlockSpec((1, 1), lambda i: (0, 0)),
        ],
        out_specs=[
            pl.BlockSpec((R, DH), lambda i: (i, 0)),
            pl.BlockSpec((R, 1), lambda i: (i, 0)),
        ],
        out_shape=[
            jax.ShapeDtypeStruct((N, DH), jnp.float32),
            jax.ShapeDtypeStruct((N, 1), jnp.float32),
        ],
    )(h1, aggp2, cnt_t, W2l.T, W2r.T, b2[None, :], W3.T, b3[None, :])

    return (outc[:, 0], h1, h2)
